# baseline (speedup 1.0000x reference)
import jax
import jax.numpy as jnp
from jax import lax
from jax.experimental import pallas as pl
from jax.experimental.pallas import tpu as pltpu
from jax.experimental.pallas import tpu_sc as plsc

N, C, H, W = 4, 96, 224, 224
P = H * W
NIMG = N * C
NWORKERS = 32
IMGS_PER_WORKER = NIMG // NWORKERS
PAIRS_PER_WORKER = IMGS_PER_WORKER // 2
CH = 896
NCH = P // CH
NGRP = NCH // 2
LANES = 16
PREP_GRID = 8
PREP_CH = P // PREP_GRID


def _prep_body(g_ref, ia_ref, ib_ref, wa_ref, wb_ref):
    g = g_ref[...]
    t = (g + 1.0) * (0.5 * (W - 1))
    t0 = jnp.floor(t)
    fr = t - t0
    om = 1.0 - fr
    vlo = ((t0 >= 0.0) & (t0 <= W - 1.0)).astype(jnp.float32)
    vhi = ((t0 >= -1.0) & (t0 <= W - 2.0)).astype(jnp.float32)
    tc = jnp.clip(t0, 0.0, W - 1.0).astype(jnp.int32)
    th = jnp.clip(t0 + 1.0, 0.0, W - 1.0).astype(jnp.int32)

    def nxt(v):
        return jnp.roll(v, -1, axis=1)

    def prv(v):
        return jnp.roll(v, 1, axis=1)

    ylo_w = nxt(tc) * W
    yhi_w = nxt(th) * W
    wy0 = nxt(om) * nxt(vlo)
    wy1 = nxt(fr) * nxt(vhi)
    wx0 = om * vlo
    wx1 = fr * vhi

    idx0 = ylo_w + tc
    idx1 = ylo_w + th
    idx2 = yhi_w + tc
    idx3 = yhi_w + th
    w0 = wx0 * wy0
    w1 = wx1 * wy0
    w2 = wx0 * wy1
    w3 = wx1 * wy1

    even = (lax.broadcasted_iota(jnp.int32, g.shape, 1) % 2) == 0
    ia_ref[...] = jnp.where(even, idx0, prv(idx1))
    ib_ref[...] = jnp.where(even, idx2, prv(idx3))
    wa_ref[...] = jnp.where(even, w0, prv(w1))
    wb_ref[...] = jnp.where(even, w2, prv(w3))


def _prep(gxy):
    blk = pl.BlockSpec((N, 2 * PREP_CH), lambda i: (0, i))
    return pl.pallas_call(
        _prep_body,
        grid=(PREP_GRID,),
        in_specs=[blk],
        out_specs=[blk] * 4,
        out_shape=[jax.ShapeDtypeStruct((N, 2 * P), jnp.int32),
                   jax.ShapeDtypeStruct((N, 2 * P), jnp.int32),
                   jax.ShapeDtypeStruct((N, 2 * P), jnp.float32),
                   jax.ShapeDtypeStruct((N, 2 * P), jnp.float32)],
    )(gxy)


def _sc_body(inp_ref, ia_ref, ib_ref, wa_ref, wb_ref, out_ref,
             img0, img1, ia0, ia1, ibb0, ibb1, wa0, wa1, wbb0, wbb1, obuf,
             semi0, semi1, semo0, semo1):
    wid = lax.axis_index("s") * 2 + lax.axis_index("c")
    n = wid // (NWORKERS // N)
    semi = (semi0, semi1)
    semo = (semo0, semo1)
    iabuf = (ia0, ia1)
    ibbuf = (ibb0, ibb1)
    wabuf = (wa0, wa1)
    wbbuf = (wbb0, wbb1)
    iota2 = lax.iota(jnp.int32, LANES) * 2

    def in_copies(b, q):
        sl = pl.ds(q * 2 * CH, 2 * CH)
        return (pltpu.make_async_copy(ia_ref.at[n, sl], iabuf[b], semi[b]),
                pltpu.make_async_copy(ib_ref.at[n, sl], ibbuf[b], semi[b]),
                pltpu.make_async_copy(wa_ref.at[n, sl], wabuf[b], semi[b]),
                pltpu.make_async_copy(wb_ref.at[n, sl], wbbuf[b], semi[b]))

    def out_copies(b, q, f0):
        sl = pl.ds(q * CH, CH)
        return (pltpu.make_async_copy(obuf.at[b, 0], out_ref.at[f0, sl], semo[b]),
                pltpu.make_async_copy(obuf.at[b, 1], out_ref.at[f0 + 1, sl], semo[b]))

    def pair_body(p, _):
        f0 = wid * IMGS_PER_WORKER + 2 * p
        pltpu.sync_copy(inp_ref.at[f0], img0)
        pltpu.sync_copy(inp_ref.at[f0 + 1], img1)

        for b in (0, 1):
            for cp in in_copies(b, b):
                cp.start()

        def group_body(g, _):
            for b in (0, 1):
                q = 2 * g + b
                for cp in in_copies(b, q):
                    cp.wait()

                @pl.when(g > 0)
                def _():
                    for cp in out_copies(b, q - 2, f0):
                        cp.wait()

                @plsc.parallel_loop(0, CH, step=LANES, unroll=4)
                def vec_body(i):
                    ev = iota2 + 2 * i
                    od = ev + 1
                    ii0 = plsc.load_gather(iabuf[b], [ev])
                    ii1 = plsc.load_gather(iabuf[b], [od])
                    ii2 = plsc.load_gather(ibbuf[b], [ev])
                    ii3 = plsc.load_gather(ibbuf[b], [od])
                    ww0 = plsc.load_gather(wabuf[b], [ev])
                    ww1 = plsc.load_gather(wabuf[b], [od])
                    ww2 = plsc.load_gather(wbbuf[b], [ev])
                    ww3 = plsc.load_gather(wbbuf[b], [od])
                    vs = pl.ds(i, LANES)
                    for s, img in ((0, img0), (1, img1)):
                        acc = (plsc.load_gather(img, [ii0]) * ww0
                               + plsc.load_gather(img, [ii1]) * ww1
                               + plsc.load_gather(img, [ii2]) * ww2
                               + plsc.load_gather(img, [ii3]) * ww3)
                        obuf[b, s, vs] = acc

                for cp in out_copies(b, q, f0):
                    cp.start()

                @pl.when(g < NGRP - 1)
                def _():
                    for cp in in_copies(b, q + 2):
                        cp.start()
            return 0

        lax.fori_loop(0, NGRP, group_body, 0)
        for b in (0, 1):
            for cp in out_copies(b, NCH - 2 + b, f0):
                cp.wait()
        return 0

    lax.fori_loop(0, PAIRS_PER_WORKER, pair_body, 0)


def _sc_sample(inp_flat, ia, ib, wa, wb):
    mesh = plsc.VectorSubcoreMesh(core_axis_name="c", subcore_axis_name="s")
    fn = pl.kernel(
        _sc_body,
        out_type=jax.ShapeDtypeStruct((NIMG, P), jnp.float32),
        mesh=mesh,
        compiler_params=pltpu.CompilerParams(needs_layout_passes=False),
        scratch_types=[
            pltpu.VMEM((P,), jnp.float32),
            pltpu.VMEM((P,), jnp.float32),
            pltpu.VMEM((2 * CH,), jnp.int32),
            pltpu.VMEM((2 * CH,), jnp.int32),
            pltpu.VMEM((2 * CH,), jnp.int32),
            pltpu.VMEM((2 * CH,), jnp.int32),
            pltpu.VMEM((2 * CH,), jnp.float32),
            pltpu.VMEM((2 * CH,), jnp.float32),
            pltpu.VMEM((2 * CH,), jnp.float32),
            pltpu.VMEM((2 * CH,), jnp.float32),
            pltpu.VMEM((2, 2, CH), jnp.float32),
            pltpu.SemaphoreType.DMA,
            pltpu.SemaphoreType.DMA,
            pltpu.SemaphoreType.DMA,
            pltpu.SemaphoreType.DMA,
        ],
    )
    return fn(inp_flat, ia, ib, wa, wb)


@jax.jit
def _run(input, grid):
    gxy = grid.reshape(N, 2 * P)
    ia, ib, wa, wb = _prep(gxy)
    inp_flat = input.reshape(NIMG, H * W)
    out_flat = _sc_sample(inp_flat, ia, ib, wa, wb)
    return out_flat.reshape(N, C, H, W)


def kernel(input, grid, interpolation_mode, padding_mode, align_corners):
    return _run(input, grid)


# speedup vs baseline: 1.0006x; 1.0006x over previous
import jax
import jax.numpy as jnp
from jax import lax
from jax.experimental import pallas as pl
from jax.experimental.pallas import tpu as pltpu
from jax.experimental.pallas import tpu_sc as plsc

N, C, H, W = 4, 96, 224, 224
P = H * W
NIMG = N * C
NWORKERS = 32
IMGS_PER_WORKER = NIMG // NWORKERS
PAIRS_PER_WORKER = IMGS_PER_WORKER // 2
CH = 896
NCH = P // CH
NGRP = NCH // 2
LANES = 16
PREP_GRID = 8
PREP_CH = P // PREP_GRID


def _prep_body(g_ref, ia_ref, ib_ref, wa_ref, wb_ref):
    g = g_ref[...]
    t = (g + 1.0) * (0.5 * (W - 1))
    t0 = jnp.floor(t)
    fr = t - t0
    om = 1.0 - fr
    vlo = ((t0 >= 0.0) & (t0 <= W - 1.0)).astype(jnp.float32)
    vhi = ((t0 >= -1.0) & (t0 <= W - 2.0)).astype(jnp.float32)
    tc = jnp.clip(t0, 0.0, W - 1.0).astype(jnp.int32)
    th = jnp.clip(t0 + 1.0, 0.0, W - 1.0).astype(jnp.int32)

    def nxt(v):
        return pltpu.roll(v, 2 * PREP_CH - 1, 1)

    def prv(v):
        return pltpu.roll(v, 1, 1)

    ylo_w = nxt(tc) * W
    yhi_w = nxt(th) * W
    wy0 = nxt(om) * nxt(vlo)
    wy1 = nxt(fr) * nxt(vhi)
    wx0 = om * vlo
    wx1 = fr * vhi

    idx0 = ylo_w + tc
    idx1 = ylo_w + th
    idx2 = yhi_w + tc
    idx3 = yhi_w + th
    w0 = wx0 * wy0
    w1 = wx1 * wy0
    w2 = wx0 * wy1
    w3 = wx1 * wy1

    even = (lax.broadcasted_iota(jnp.int32, g.shape, 1) % 2) == 0
    ia_ref[...] = jnp.where(even, idx0, prv(idx1))
    ib_ref[...] = jnp.where(even, idx2, prv(idx3))
    wa_ref[...] = jnp.where(even, w0, prv(w1))
    wb_ref[...] = jnp.where(even, w2, prv(w3))


def _prep(gxy):
    blk = pl.BlockSpec((N, 2 * PREP_CH), lambda i: (0, i))
    return pl.pallas_call(
        _prep_body,
        grid=(PREP_GRID,),
        in_specs=[blk],
        out_specs=[blk] * 4,
        out_shape=[jax.ShapeDtypeStruct((N, 2 * P), jnp.int32),
                   jax.ShapeDtypeStruct((N, 2 * P), jnp.int32),
                   jax.ShapeDtypeStruct((N, 2 * P), jnp.float32),
                   jax.ShapeDtypeStruct((N, 2 * P), jnp.float32)],
    )(gxy)


def _sc_body(inp_ref, ia_ref, ib_ref, wa_ref, wb_ref, out_ref,
             img0, img1, ia0, ia1, ibb0, ibb1, wa0, wa1, wbb0, wbb1, obuf,
             semi0, semi1, semo0, semo1):
    wid = lax.axis_index("s") * 2 + lax.axis_index("c")
    n = wid // (NWORKERS // N)
    semi = (semi0, semi1)
    semo = (semo0, semo1)
    iabuf = (ia0, ia1)
    ibbuf = (ibb0, ibb1)
    wabuf = (wa0, wa1)
    wbbuf = (wbb0, wbb1)
    iota2 = lax.iota(jnp.int32, LANES) * 2

    def in_copies(b, q):
        sl = pl.ds(q * 2 * CH, 2 * CH)
        return (pltpu.make_async_copy(ia_ref.at[n, sl], iabuf[b], semi[b]),
                pltpu.make_async_copy(ib_ref.at[n, sl], ibbuf[b], semi[b]),
                pltpu.make_async_copy(wa_ref.at[n, sl], wabuf[b], semi[b]),
                pltpu.make_async_copy(wb_ref.at[n, sl], wbbuf[b], semi[b]))

    def out_copies(b, q, f0):
        sl = pl.ds(q * CH, CH)
        return (pltpu.make_async_copy(obuf.at[b, 0], out_ref.at[f0, sl], semo[b]),
                pltpu.make_async_copy(obuf.at[b, 1], out_ref.at[f0 + 1, sl], semo[b]))

    def pair_body(p, _):
        f0 = wid * IMGS_PER_WORKER + 2 * p
        pltpu.sync_copy(inp_ref.at[f0], img0)
        pltpu.sync_copy(inp_ref.at[f0 + 1], img1)

        for b in (0, 1):
            for cp in in_copies(b, b):
                cp.start()

        def group_body(g, _):
            for b in (0, 1):
                q = 2 * g + b
                for cp in in_copies(b, q):
                    cp.wait()

                @pl.when(g > 0)
                def _():
                    for cp in out_copies(b, q - 2, f0):
                        cp.wait()

                @plsc.parallel_loop(0, CH, step=LANES, unroll=4)
                def vec_body(i):
                    ev = iota2 + 2 * i
                    od = ev + 1
                    ii0 = plsc.load_gather(iabuf[b], [ev])
                    ii1 = plsc.load_gather(iabuf[b], [od])
                    ii2 = plsc.load_gather(ibbuf[b], [ev])
                    ii3 = plsc.load_gather(ibbuf[b], [od])
                    ww0 = plsc.load_gather(wabuf[b], [ev])
                    ww1 = plsc.load_gather(wabuf[b], [od])
                    ww2 = plsc.load_gather(wbbuf[b], [ev])
                    ww3 = plsc.load_gather(wbbuf[b], [od])
                    vs = pl.ds(i, LANES)
                    for s, img in ((0, img0), (1, img1)):
                        acc = (plsc.load_gather(img, [ii0]) * ww0
                               + plsc.load_gather(img, [ii1]) * ww1
                               + plsc.load_gather(img, [ii2]) * ww2
                               + plsc.load_gather(img, [ii3]) * ww3)
                        obuf[b, s, vs] = acc

                for cp in out_copies(b, q, f0):
                    cp.start()

                @pl.when(g < NGRP - 1)
                def _():
                    for cp in in_copies(b, q + 2):
                        cp.start()
            return 0

        lax.fori_loop(0, NGRP, group_body, 0)
        for b in (0, 1):
            for cp in out_copies(b, NCH - 2 + b, f0):
                cp.wait()
        return 0

    lax.fori_loop(0, PAIRS_PER_WORKER, pair_body, 0)


def _sc_sample(inp_flat, ia, ib, wa, wb):
    mesh = plsc.VectorSubcoreMesh(core_axis_name="c", subcore_axis_name="s")
    fn = pl.kernel(
        _sc_body,
        out_type=jax.ShapeDtypeStruct((NIMG, P), jnp.float32),
        mesh=mesh,
        compiler_params=pltpu.CompilerParams(needs_layout_passes=False),
        scratch_types=[
            pltpu.VMEM((P,), jnp.float32),
            pltpu.VMEM((P,), jnp.float32),
            pltpu.VMEM((2 * CH,), jnp.int32),
            pltpu.VMEM((2 * CH,), jnp.int32),
            pltpu.VMEM((2 * CH,), jnp.int32),
            pltpu.VMEM((2 * CH,), jnp.int32),
            pltpu.VMEM((2 * CH,), jnp.float32),
            pltpu.VMEM((2 * CH,), jnp.float32),
            pltpu.VMEM((2 * CH,), jnp.float32),
            pltpu.VMEM((2 * CH,), jnp.float32),
            pltpu.VMEM((2, 2, CH), jnp.float32),
            pltpu.SemaphoreType.DMA,
            pltpu.SemaphoreType.DMA,
            pltpu.SemaphoreType.DMA,
            pltpu.SemaphoreType.DMA,
        ],
    )
    return fn(inp_flat, ia, ib, wa, wb)


@jax.jit
def _run(input, grid):
    gxy = grid.reshape(N, 2 * P)
    ia, ib, wa, wb = _prep(gxy)
    inp_flat = input.reshape(NIMG, H * W)
    out_flat = _sc_sample(inp_flat, ia, ib, wa, wb)
    return out_flat.reshape(N, C, H, W)


def kernel(input, grid, interpolation_mode, padding_mode, align_corners):
    return _run(input, grid)


# speedup vs baseline: 1.1919x; 1.1912x over previous
import jax
import jax.numpy as jnp
from jax import lax
from jax.experimental import pallas as pl
from jax.experimental.pallas import tpu as pltpu
from jax.experimental.pallas import tpu_sc as plsc

N, C, H, W = 4, 96, 224, 224
P = H * W
NIMG = N * C
NWORKERS = 32
IMGS_PER_WORKER = NIMG // NWORKERS
PAIRS_PER_WORKER = IMGS_PER_WORKER // 2
CH = 896
NCH = P // CH
NGRP = NCH // 2
LANES = 16
PREP_GRID = 8
PREP_CH = P // PREP_GRID


def _pack_bf16(lo, hi):
    u_lo = lax.bitcast_convert_type(lo.astype(jnp.bfloat16), jnp.uint16)
    u_hi = lax.bitcast_convert_type(hi.astype(jnp.bfloat16), jnp.uint16)
    return (u_lo.astype(jnp.int32) | (u_hi.astype(jnp.int32) << 16))


def _prep_body(gx_ref, gy_ref, ia_ref, ib_ref, wx_ref, wy_ref):
    gx = gx_ref[...]
    gy = gy_ref[...]
    ix = (gx + 1.0) * (0.5 * (W - 1))
    iy = (gy + 1.0) * (0.5 * (H - 1))
    ix0 = jnp.floor(ix)
    iy0 = jnp.floor(iy)
    wx1 = ix - ix0
    wx0 = 1.0 - wx1
    wy1 = iy - iy0
    wy0 = 1.0 - wy1

    def lohi(t0, frac_hi):
        vlo = ((t0 >= 0.0) & (t0 <= W - 1.0)).astype(jnp.float32)
        vhi = ((t0 >= -1.0) & (t0 <= W - 2.0)).astype(jnp.float32)
        clo = jnp.clip(t0, 0.0, W - 1.0).astype(jnp.int32)
        chi = jnp.clip(t0 + 1.0, 0.0, W - 1.0).astype(jnp.int32)
        return clo, chi, (1.0 - frac_hi) * vlo, frac_hi * vhi

    x_lo, x_hi, X0, X1 = lohi(ix0, wx1)
    y_lo, y_hi, Y0, Y1 = lohi(iy0, wy1)

    ia_ref[...] = (y_lo * W + x_lo) | ((y_lo * W + x_hi) << 16)
    ib_ref[...] = (y_hi * W + x_lo) | ((y_hi * W + x_hi) << 16)
    wx_ref[...] = _pack_bf16(X0, X1)
    wy_ref[...] = _pack_bf16(Y0, Y1)


def _prep(gx, gy):
    blk = pl.BlockSpec((N, PREP_CH), lambda i: (0, i))
    return pl.pallas_call(
        _prep_body,
        grid=(PREP_GRID,),
        in_specs=[blk, blk],
        out_specs=[blk] * 4,
        out_shape=[jax.ShapeDtypeStruct((N, P), jnp.int32)] * 4,
    )(gx, gy)


def _sc_body(inp_ref, ia_ref, ib_ref, wx_ref, wy_ref, out_ref,
             img0, img1, ia0, ia1, ibb0, ibb1, wx0, wx1_, wy0, wy1_, obuf,
             semi0, semi1, semo0, semo1):
    wid = lax.axis_index("s") * 2 + lax.axis_index("c")
    n = wid // (NWORKERS // N)
    semi = (semi0, semi1)
    semo = (semo0, semo1)
    iabuf = (ia0, ia1)
    ibbuf = (ibb0, ibb1)
    wxbuf = (wx0, wx1_)
    wybuf = (wy0, wy1_)

    def in_copies(b, q):
        sl = pl.ds(q * CH, CH)
        return (pltpu.make_async_copy(ia_ref.at[n, sl], iabuf[b], semi[b]),
                pltpu.make_async_copy(ib_ref.at[n, sl], ibbuf[b], semi[b]),
                pltpu.make_async_copy(wx_ref.at[n, sl], wxbuf[b], semi[b]),
                pltpu.make_async_copy(wy_ref.at[n, sl], wybuf[b], semi[b]))

    def out_copies(b, q, f0):
        sl = pl.ds(q * CH, CH)
        return (pltpu.make_async_copy(obuf.at[b, 0], out_ref.at[f0, sl], semo[b]),
                pltpu.make_async_copy(obuf.at[b, 1], out_ref.at[f0 + 1, sl], semo[b]))

    def pair_body(p, _):
        f0 = wid * IMGS_PER_WORKER + 2 * p
        pltpu.sync_copy(inp_ref.at[f0], img0)
        pltpu.sync_copy(inp_ref.at[f0 + 1], img1)

        for b in (0, 1):
            for cp in in_copies(b, b):
                cp.start()

        def group_body(g, _):
            for b in (0, 1):
                q = 2 * g + b
                for cp in in_copies(b, q):
                    cp.wait()

                @pl.when(g > 0)
                def _():
                    for cp in out_copies(b, q - 2, f0):
                        cp.wait()

                @plsc.parallel_loop(0, CH, step=LANES, unroll=4)
                def vec_body(i):
                    vs = pl.ds(i, LANES)
                    pia = iabuf[b][vs]
                    pib = ibbuf[b][vs]
                    pwx = wxbuf[b][vs]
                    pwy = wybuf[b][vs]
                    mask = jnp.full((LANES,), 0xFFFF, jnp.int32)
                    hmask = jnp.full((LANES,), -65536, jnp.int32)
                    i00 = pia & mask
                    i10 = lax.shift_right_logical(pia, 16)
                    i01 = pib & mask
                    i11 = lax.shift_right_logical(pib, 16)
                    X0 = plsc.bitcast(lax.shift_left(pwx, 16), jnp.float32)
                    X1 = plsc.bitcast(pwx & hmask, jnp.float32)
                    Y0 = plsc.bitcast(lax.shift_left(pwy, 16), jnp.float32)
                    Y1 = plsc.bitcast(pwy & hmask, jnp.float32)
                    for s, img in ((0, img0), (1, img1)):
                        r0 = (plsc.load_gather(img, [i00]) * X0
                              + plsc.load_gather(img, [i10]) * X1)
                        r1 = (plsc.load_gather(img, [i01]) * X0
                              + plsc.load_gather(img, [i11]) * X1)
                        obuf[b, s, vs] = r0 * Y0 + r1 * Y1

                for cp in out_copies(b, q, f0):
                    cp.start()

                @pl.when(g < NGRP - 1)
                def _():
                    for cp in in_copies(b, q + 2):
                        cp.start()
            return 0

        lax.fori_loop(0, NGRP, group_body, 0)
        for b in (0, 1):
            for cp in out_copies(b, NCH - 2 + b, f0):
                cp.wait()
        return 0

    lax.fori_loop(0, PAIRS_PER_WORKER, pair_body, 0)


def _sc_sample(inp_flat, ia, ib, wx, wy):
    mesh = plsc.VectorSubcoreMesh(core_axis_name="c", subcore_axis_name="s")
    fn = pl.kernel(
        _sc_body,
        out_type=jax.ShapeDtypeStruct((NIMG, P), jnp.float32),
        mesh=mesh,
        compiler_params=pltpu.CompilerParams(needs_layout_passes=False),
        scratch_types=[
            pltpu.VMEM((P,), jnp.float32),
            pltpu.VMEM((P,), jnp.float32),
            pltpu.VMEM((CH,), jnp.int32),
            pltpu.VMEM((CH,), jnp.int32),
            pltpu.VMEM((CH,), jnp.int32),
            pltpu.VMEM((CH,), jnp.int32),
            pltpu.VMEM((CH,), jnp.int32),
            pltpu.VMEM((CH,), jnp.int32),
            pltpu.VMEM((CH,), jnp.int32),
            pltpu.VMEM((CH,), jnp.int32),
            pltpu.VMEM((2, 2, CH), jnp.float32),
            pltpu.SemaphoreType.DMA,
            pltpu.SemaphoreType.DMA,
            pltpu.SemaphoreType.DMA,
            pltpu.SemaphoreType.DMA,
        ],
    )
    return fn(inp_flat, ia, ib, wx, wy)


@jax.jit
def _run(input, grid):
    gxy = jnp.moveaxis(grid.reshape(N, P, 2), 2, 1)
    ia, ib, wx, wy = _prep(gxy[:, 0], gxy[:, 1])
    inp_flat = input.reshape(NIMG, H * W)
    out_flat = _sc_sample(inp_flat, ia, ib, wx, wy)
    return out_flat.reshape(N, C, H, W)


def kernel(input, grid, interpolation_mode, padding_mode, align_corners):
    return _run(input, grid)


# speedup vs baseline: 1.2027x; 1.0091x over previous
import jax
import jax.numpy as jnp
from jax import lax
from jax.experimental import pallas as pl
from jax.experimental.pallas import tpu as pltpu
from jax.experimental.pallas import tpu_sc as plsc

N, C, H, W = 4, 96, 224, 224
P = H * W
NIMG = N * C
NWORKERS = 32
IMGS_PER_WORKER = NIMG // NWORKERS
PAIRS_PER_WORKER = IMGS_PER_WORKER // 2
CH = 1792
NCH = P // CH
NGRP = NCH // 2
LANES = 16
PREP_GRID = 8
PREP_CH = P // PREP_GRID


def _pack_bf16(lo, hi):
    u_lo = lax.bitcast_convert_type(lo.astype(jnp.bfloat16), jnp.uint16)
    u_hi = lax.bitcast_convert_type(hi.astype(jnp.bfloat16), jnp.uint16)
    return (u_lo.astype(jnp.int32) | (u_hi.astype(jnp.int32) << 16))


def _prep_body(gx_ref, gy_ref, ia_ref, ib_ref, wx_ref, wy_ref):
    gx = gx_ref[...]
    gy = gy_ref[...]
    ix = (gx + 1.0) * (0.5 * (W - 1))
    iy = (gy + 1.0) * (0.5 * (H - 1))
    ix0 = jnp.floor(ix)
    iy0 = jnp.floor(iy)
    wx1 = ix - ix0
    wx0 = 1.0 - wx1
    wy1 = iy - iy0
    wy0 = 1.0 - wy1

    def lohi(t0, frac_hi):
        vlo = ((t0 >= 0.0) & (t0 <= W - 1.0)).astype(jnp.float32)
        vhi = ((t0 >= -1.0) & (t0 <= W - 2.0)).astype(jnp.float32)
        clo = jnp.clip(t0, 0.0, W - 1.0).astype(jnp.int32)
        chi = jnp.clip(t0 + 1.0, 0.0, W - 1.0).astype(jnp.int32)
        return clo, chi, (1.0 - frac_hi) * vlo, frac_hi * vhi

    x_lo, x_hi, X0, X1 = lohi(ix0, wx1)
    y_lo, y_hi, Y0, Y1 = lohi(iy0, wy1)

    ia_ref[...] = (y_lo * W + x_lo) | ((y_lo * W + x_hi) << 16)
    ib_ref[...] = (y_hi * W + x_lo) | ((y_hi * W + x_hi) << 16)
    wx_ref[...] = _pack_bf16(X0, X1)
    wy_ref[...] = _pack_bf16(Y0, Y1)


def _prep(gx, gy):
    blk = pl.BlockSpec((N, PREP_CH), lambda i: (0, i))
    return pl.pallas_call(
        _prep_body,
        grid=(PREP_GRID,),
        in_specs=[blk, blk],
        out_specs=[blk] * 4,
        out_shape=[jax.ShapeDtypeStruct((N, P), jnp.int32)] * 4,
    )(gx, gy)


def _sc_body(inp_ref, ia_ref, ib_ref, wx_ref, wy_ref, out_ref,
             img0, img1, ia0, ia1, ibb0, ibb1, wx0, wx1_, wy0, wy1_, obuf,
             semi0, semi1, semo0, semo1):
    wid = lax.axis_index("s") * 2 + lax.axis_index("c")
    n = wid // (NWORKERS // N)
    semi = (semi0, semi1)
    semo = (semo0, semo1)
    iabuf = (ia0, ia1)
    ibbuf = (ibb0, ibb1)
    wxbuf = (wx0, wx1_)
    wybuf = (wy0, wy1_)

    def in_copies(b, q):
        sl = pl.ds(q * CH, CH)
        return (pltpu.make_async_copy(ia_ref.at[n, sl], iabuf[b], semi[b]),
                pltpu.make_async_copy(ib_ref.at[n, sl], ibbuf[b], semi[b]),
                pltpu.make_async_copy(wx_ref.at[n, sl], wxbuf[b], semi[b]),
                pltpu.make_async_copy(wy_ref.at[n, sl], wybuf[b], semi[b]))

    def out_copies(b, q, f0):
        sl = pl.ds(q * CH, CH)
        return (pltpu.make_async_copy(obuf.at[b, 0], out_ref.at[f0, sl], semo[b]),
                pltpu.make_async_copy(obuf.at[b, 1], out_ref.at[f0 + 1, sl], semo[b]))

    def pair_body(p, _):
        f0 = wid * IMGS_PER_WORKER + 2 * p
        pltpu.sync_copy(inp_ref.at[f0], img0)
        pltpu.sync_copy(inp_ref.at[f0 + 1], img1)

        for b in (0, 1):
            for cp in in_copies(b, b):
                cp.start()

        def group_body(g, _):
            for b in (0, 1):
                q = 2 * g + b
                for cp in in_copies(b, q):
                    cp.wait()

                @pl.when(g > 0)
                def _():
                    for cp in out_copies(b, q - 2, f0):
                        cp.wait()

                @plsc.parallel_loop(0, CH, step=LANES, unroll=4)
                def vec_body(i):
                    vs = pl.ds(i, LANES)
                    pia = iabuf[b][vs]
                    pib = ibbuf[b][vs]
                    pwx = wxbuf[b][vs]
                    pwy = wybuf[b][vs]
                    mask = jnp.full((LANES,), 0xFFFF, jnp.int32)
                    hmask = jnp.full((LANES,), -65536, jnp.int32)
                    i00 = pia & mask
                    i10 = lax.shift_right_logical(pia, 16)
                    i01 = pib & mask
                    i11 = lax.shift_right_logical(pib, 16)
                    X0 = plsc.bitcast(lax.shift_left(pwx, 16), jnp.float32)
                    X1 = plsc.bitcast(pwx & hmask, jnp.float32)
                    Y0 = plsc.bitcast(lax.shift_left(pwy, 16), jnp.float32)
                    Y1 = plsc.bitcast(pwy & hmask, jnp.float32)
                    for s, img in ((0, img0), (1, img1)):
                        r0 = (plsc.load_gather(img, [i00]) * X0
                              + plsc.load_gather(img, [i10]) * X1)
                        r1 = (plsc.load_gather(img, [i01]) * X0
                              + plsc.load_gather(img, [i11]) * X1)
                        obuf[b, s, vs] = r0 * Y0 + r1 * Y1

                for cp in out_copies(b, q, f0):
                    cp.start()

                @pl.when(g < NGRP - 1)
                def _():
                    for cp in in_copies(b, q + 2):
                        cp.start()
            return 0

        lax.fori_loop(0, NGRP, group_body, 0)
        for b in (0, 1):
            for cp in out_copies(b, NCH - 2 + b, f0):
                cp.wait()
        return 0

    lax.fori_loop(0, PAIRS_PER_WORKER, pair_body, 0)


def _sc_sample(inp_flat, ia, ib, wx, wy):
    mesh = plsc.VectorSubcoreMesh(core_axis_name="c", subcore_axis_name="s")
    fn = pl.kernel(
        _sc_body,
        out_type=jax.ShapeDtypeStruct((NIMG, P), jnp.float32),
        mesh=mesh,
        compiler_params=pltpu.CompilerParams(needs_layout_passes=False),
        scratch_types=[
            pltpu.VMEM((P,), jnp.float32),
            pltpu.VMEM((P,), jnp.float32),
            pltpu.VMEM((CH,), jnp.int32),
            pltpu.VMEM((CH,), jnp.int32),
            pltpu.VMEM((CH,), jnp.int32),
            pltpu.VMEM((CH,), jnp.int32),
            pltpu.VMEM((CH,), jnp.int32),
            pltpu.VMEM((CH,), jnp.int32),
            pltpu.VMEM((CH,), jnp.int32),
            pltpu.VMEM((CH,), jnp.int32),
            pltpu.VMEM((2, 2, CH), jnp.float32),
            pltpu.SemaphoreType.DMA,
            pltpu.SemaphoreType.DMA,
            pltpu.SemaphoreType.DMA,
            pltpu.SemaphoreType.DMA,
        ],
    )
    return fn(inp_flat, ia, ib, wx, wy)


@jax.jit
def _run(input, grid):
    gxy = jnp.moveaxis(grid.reshape(N, P, 2), 2, 1)
    ia, ib, wx, wy = _prep(gxy[:, 0], gxy[:, 1])
    inp_flat = input.reshape(NIMG, H * W)
    out_flat = _sc_sample(inp_flat, ia, ib, wx, wy)
    return out_flat.reshape(N, C, H, W)


def kernel(input, grid, interpolation_mode, padding_mode, align_corners):
    return _run(input, grid)
